# baseline (speedup 1.0000x reference)
import jax
import jax.numpy as jnp
from jax import lax
from jax.experimental import pallas as pl
from jax.experimental.pallas import tpu as pltpu
from jax.experimental.pallas import tpu_sc as plsc

_N_EMB = 8192
_D = 256
_ALPHA = -5.0
_BK = 1024
_NK = _N_EMB // _BK
_N = 2304
_NC = 2
_NS = 16
_BPW = _N // (_NC * _NS)
_PREC = jax.lax.Precision.DEFAULT


def _pass0_kernel(x_ref, emb_ref, idx_ref, w_ref, commit_ref, xt_ref,
                  ae2_ref, l_ref):
    j = pl.program_id(0)
    e = emb_ref[...]

    @pl.when(j == 0)
    def _init():
        x = x_ref[...]
        xt_ref[...] = x.T
        commit_ref[...] = jnp.sum(x * x) * jnp.ones((1, 1), jnp.float32)
        w_ref[...] = jnp.full((1, _N), -jnp.inf, dtype=jnp.float32)
        l_ref[...] = jnp.zeros((1, _N), dtype=jnp.float32)
        idx_ref[...] = jnp.zeros((1, _N), dtype=jnp.int32)

    ae2 = _ALPHA * jnp.sum(e * e, axis=1, keepdims=True)
    ae2_ref[...] = ae2
    xe = jnp.dot(e, xt_ref[...], preferred_element_type=jnp.float32,
                 precision=_PREC)
    d = ae2 + (-2.0 * _ALPHA) * xe

    bm = jnp.max(d, axis=0, keepdims=True)
    iota = jax.lax.broadcasted_iota(jnp.int32, (_BK, _N), 0)
    barg = jnp.min(jnp.where(d == bm, iota, _N_EMB), axis=0,
                   keepdims=True) + j * _BK
    m_old = w_ref[...]
    m_new = jnp.maximum(m_old, bm)
    l_ref[...] = (l_ref[...] * jnp.exp(m_old - m_new)
                  + jnp.sum(jnp.exp(d - m_new), axis=0, keepdims=True))
    w_ref[...] = m_new
    idx_ref[...] = jnp.where(bm > m_old, barg, idx_ref[...])

    @pl.when(j == _NK - 1)
    def _fini():
        commit_ref[...] = (jnp.sum(w_ref[...]) / _ALPHA + commit_ref[...]) \
            / (_N * _D)
        w_ref[...] = w_ref[...] + jnp.log(l_ref[...])


def _pass1_kernel(xt_ref, emb_ref, ae2_ref, w_ref, counts_ref, code_ref,
                  prob_ref):
    j = pl.program_id(0)
    e = emb_ref[...]

    @pl.when(j == 0)
    def _init():
        prob_ref[...] = jnp.zeros((1, 1), dtype=jnp.float32)

    xe = jnp.dot(e, xt_ref[...], preferred_element_type=jnp.float32,
                 precision=_PREC)
    d = ae2_ref[...] + (-2.0 * _ALPHA) * xe

    pb = jnp.exp(d - w_ref[...])
    col = jnp.sum(pb, axis=1, keepdims=True) / _N
    prob_ref[...] = prob_ref[...] - jnp.sum(col * jnp.log2(col + 1e-10))

    @pl.when(j == _NK - 1)
    def _fini():
        hp = (counts_ref[0:1, :] + counts_ref[1:2, :]) / _N
        code_ref[...] = -jnp.sum(hp * jnp.log2(hp + 1e-10)) \
            * jnp.ones((1, 1), jnp.float32)


def _sc_gather_hist(emb, idx_flat, ones, zeros):
    mesh = plsc.VectorSubcoreMesh(core_axis_name="c", subcore_axis_name="s")

    @pl.kernel(
        out_type=[jax.ShapeDtypeStruct((_N, _D), jnp.float32),
                  jax.ShapeDtypeStruct((_NC, _N_EMB), jnp.float32)],
        mesh=mesh,
        scratch_types=[
            pltpu.VMEM((_BPW,), jnp.int32),
            pltpu.VMEM((_BPW, _D), jnp.float32),
            pltpu.VMEM((_BPW,), jnp.float32),
            pltpu.VMEM_SHARED((_N_EMB,), jnp.float32),
            pltpu.SemaphoreType.DMA,
        ],
    )
    def k(emb_hbm, idx_hbm, ones_hbm, zeros_hbm, out_hbm, cnt_hbm,
          idx_v, rows_v, ones_v, hist_sh, sem):
        cid = lax.axis_index("c")
        sid = lax.axis_index("s")
        wid = cid * _NS + sid
        base = wid * _BPW

        @pl.when(sid == 0)
        def _():
            pltpu.sync_copy(zeros_hbm, hist_sh)

        pltpu.sync_copy(idx_hbm.at[pl.ds(base, _BPW)], idx_v)
        pltpu.sync_copy(ones_hbm.at[pl.ds(base, _BPW)], ones_v)
        pltpu.async_copy(emb_hbm.at[idx_v], rows_v, sem).wait()
        pltpu.sync_copy(rows_v, out_hbm.at[pl.ds(base, _BPW)])
        plsc.subcore_barrier()
        pltpu.sync_copy(ones_v, hist_sh.at[idx_v], add=True)
        plsc.subcore_barrier()

        @pl.when(sid == 0)
        def _():
            pltpu.sync_copy(hist_sh, cnt_hbm.at[cid])

    return k(emb, idx_flat, ones, zeros)


def kernel(x, embedding):
    bsz, tsz, csz = x.shape
    x_flat = x.reshape(-1, csz)
    emb = embedding.reshape(_N_EMB, _D)

    idx, w, commit, xt, ae2 = pl.pallas_call(
        _pass0_kernel,
        grid=(_NK,),
        in_specs=[
            pl.BlockSpec((_N, _D), lambda j: (0, 0)),
            pl.BlockSpec((_BK, _D), lambda j: (j, 0)),
        ],
        out_specs=[
            pl.BlockSpec((1, _N), lambda j: (0, 0)),
            pl.BlockSpec((1, _N), lambda j: (0, 0)),
            pl.BlockSpec((1, 1), lambda j: (0, 0)),
            pl.BlockSpec((_D, _N), lambda j: (0, 0)),
            pl.BlockSpec((_BK, 1), lambda j: (j, 0)),
        ],
        out_shape=[
            jax.ShapeDtypeStruct((1, _N), jnp.int32),
            jax.ShapeDtypeStruct((1, _N), jnp.float32),
            jax.ShapeDtypeStruct((1, 1), jnp.float32),
            jax.ShapeDtypeStruct((_D, _N), jnp.float32),
            jax.ShapeDtypeStruct((_N_EMB, 1), jnp.float32),
        ],
        scratch_shapes=[pltpu.VMEM((1, _N), jnp.float32)],
    )(x_flat, emb)

    quant, counts = _sc_gather_hist(
        emb, idx.reshape(_N),
        jnp.ones((_N,), jnp.float32), jnp.zeros((_N_EMB,), jnp.float32))

    code, prob = pl.pallas_call(
        _pass1_kernel,
        grid=(_NK,),
        in_specs=[
            pl.BlockSpec((_D, _N), lambda j: (0, 0)),
            pl.BlockSpec((_BK, _D), lambda j: (j, 0)),
            pl.BlockSpec((_BK, 1), lambda j: (j, 0)),
            pl.BlockSpec((1, _N), lambda j: (0, 0)),
            pl.BlockSpec((_NC, _N_EMB), lambda j: (0, 0)),
        ],
        out_specs=[
            pl.BlockSpec((1, 1), lambda j: (0, 0)),
            pl.BlockSpec((1, 1), lambda j: (0, 0)),
        ],
        out_shape=[
            jax.ShapeDtypeStruct((1, 1), jnp.float32),
            jax.ShapeDtypeStruct((1, 1), jnp.float32),
        ],
    )(xt, emb, ae2, w, counts)

    quantized = quant.reshape(bsz, tsz, csz)
    quantization_inds = idx.reshape(bsz, tsz, 1)
    return (quantized, code[0, 0], prob[0, 0], quantization_inds,
            commit[0, 0])
